# Initial kernel scaffold; baseline (speedup 1.0000x reference)
import functools

import jax
import jax.numpy as jnp
from jax import lax
from jax.experimental import pallas as pl
from jax.experimental.pallas import tpu as pltpu
from jax.experimental.pallas import tpu_sc as plsc

N_NODES = 10000
D = 128
NC = 2
NS = 16
NW = NC * NS
C = 128
J = 80
N_PAD = 10240
ROWS_PER_TILE = N_PAD // NS


def _sc_agg_body(table, src3, dst3, agg_out, deg_out,
                 agg_sh, src_v, dst_v, rows, deg_v, zbuf, sem):
    cid = lax.axis_index("c")
    sid = lax.axis_index("s")
    wid = sid * NC + cid

    zeros16 = jnp.zeros((16,), jnp.float32)

    @pl.loop(0, 128)
    def _(i):
        for k in range(8):
            zbuf[i, pl.ds(k * 16, 16)] = zeros16

    base = sid * ROWS_PER_TILE
    for r in range(ROWS_PER_TILE // 128):
        pltpu.sync_copy(zbuf, agg_sh.at[pl.ds(base + r * 128, 128)])

    @pl.loop(0, N_PAD // 16)
    def _(i):
        deg_v[pl.ds(i * 16, 16)] = zeros16

    plsc.subcore_barrier()

    pltpu.sync_copy(src3.at[wid], src_v)
    pltpu.sync_copy(dst3.at[wid], dst_v)

    ones16 = jnp.ones((16,), jnp.float32)

    @pl.loop(0, J)
    def _(j):
        pltpu.async_copy(table.at[src_v.at[j]], rows, sem).wait()
        pltpu.sync_copy(rows, agg_sh.at[dst_v.at[j]], add=True)
        for k in range(C // 16):
            idx = dst_v[j, pl.ds(k * 16, 16)]
            plsc.addupdate_scatter(deg_v, [idx], ones16)

    plsc.subcore_barrier()

    pltpu.sync_copy(agg_sh.at[pl.ds(base, ROWS_PER_TILE)],
                    agg_out.at[cid, pl.ds(base, ROWS_PER_TILE)])
    pltpu.sync_copy(deg_v, deg_out.at[wid])


@jax.jit
def _sc_agg(table, src3, dst3):
    mesh = plsc.VectorSubcoreMesh(core_axis_name="c", subcore_axis_name="s")
    return pl.kernel(
        _sc_agg_body,
        out_type=(
            jax.ShapeDtypeStruct((NC, N_PAD, D), jnp.float32),
            jax.ShapeDtypeStruct((NW, N_PAD), jnp.float32),
        ),
        mesh=mesh,
        scratch_types=[
            pltpu.VMEM_SHARED((N_PAD, D), jnp.float32),
            pltpu.VMEM((J, C), jnp.int32),
            pltpu.VMEM((J, C), jnp.int32),
            pltpu.VMEM((C, D), jnp.float32),
            pltpu.VMEM((N_PAD,), jnp.float32),
            pltpu.VMEM((128, D), jnp.float32),
            pltpu.SemaphoreType.DMA,
        ],
    )(table, src3, dst3)


def _tc_layer_body(relu, p_ref, deg_ref, x_ref, wl_ref, wr_ref, b_ref, o_ref):
    agg = p_ref[0] + p_ref[1]
    deg = jnp.sum(deg_ref[...], axis=0)
    deg = jnp.maximum(deg, 1.0)
    agg = agg / deg[:, None]
    out = (jnp.dot(agg, wl_ref[...], preferred_element_type=jnp.float32)
           + jnp.dot(x_ref[...], wr_ref[...], preferred_element_type=jnp.float32)
           + b_ref[...])
    if relu:
        out = jnp.maximum(out, 0.0)
    o_ref[...] = out


BLK = 512


@functools.partial(jax.jit, static_argnames=("relu",))
def _tc_layer(P, degP, x, W_l, W_r, b, relu):
    grid = N_PAD // BLK
    return pl.pallas_call(
        functools.partial(_tc_layer_body, relu),
        grid=(grid,),
        in_specs=[
            pl.BlockSpec((NC, BLK, D), lambda i: (0, i, 0)),
            pl.BlockSpec((NW, BLK), lambda i: (0, i)),
            pl.BlockSpec((BLK, D), lambda i: (i, 0)),
            pl.BlockSpec((D, D), lambda i: (0, 0)),
            pl.BlockSpec((D, D), lambda i: (0, 0)),
            pl.BlockSpec((1, D), lambda i: (0, 0)),
        ],
        out_specs=pl.BlockSpec((BLK, D), lambda i: (i, 0)),
        out_shape=jax.ShapeDtypeStruct((N_PAD, D), jnp.float32),
    )(P, degP, x, W_l, W_r, b.reshape(1, D))


def kernel(x, edge_index, W_l1, W_r1, b1, W_l2, W_r2, b2):
    src = edge_index[0]
    dst = edge_index[1]
    e = src.shape[0]
    pad = NW * J * C - e
    src_p = jnp.concatenate([src, jnp.zeros((pad,), jnp.int32)])
    dst_p = jnp.concatenate([dst, jnp.full((pad,), N_NODES, jnp.int32)])
    src3 = src_p.reshape(NW, J, C)
    dst3 = dst_p.reshape(NW, J, C)
    x_pad = jnp.pad(x, ((0, N_PAD - N_NODES), (0, 0)))

    P1, degP = _sc_agg(x_pad, src3, dst3)
    h = _tc_layer(P1, degP, x_pad, W_l1, W_r1, b1, relu=True)
    P2, _ = _sc_agg(h, src3, dst3)
    out = _tc_layer(P2, degP, h, W_l2, W_r2, b2, relu=False)
    return out[:N_NODES]


# speedup vs baseline: 3.0564x; 3.0564x over previous
import functools

import jax
import jax.numpy as jnp
from jax import lax
from jax.experimental import pallas as pl
from jax.experimental.pallas import tpu as pltpu
from jax.experimental.pallas import tpu_sc as plsc

N_NODES = 10000
D = 128
NC = 2
NS = 16
NW = NC * NS
C = 128
J = 80
N_PAD = 10240
ROWS_PER_TILE = N_PAD // NS
DW = 16


def _sc_agg_body(table, src3, dst3, agg_out, deg_out,
                 agg_sh, src_b, dst_b, rows, deg_v, sem):
    cid = lax.axis_index("c")
    sid = lax.axis_index("s")
    wid = sid * NC + cid

    zeros16 = jnp.zeros((16,), jnp.float32)
    ones16 = jnp.ones((16,), jnp.float32)

    @pl.loop(0, C)
    def _(i):
        for k in range(8):
            rows[i, pl.ds(k * 16, 16)] = zeros16

    @pl.loop(0, N_PAD // 16)
    def _(i):
        deg_v[pl.ds(i * 16, 16)] = zeros16

    base = sid * ROWS_PER_TILE
    for r in range(ROWS_PER_TILE // C):
        pltpu.sync_copy(rows, agg_sh.at[pl.ds(base + r * C, C)])

    plsc.subcore_barrier()

    @pl.loop(0, J)
    def _(j):
        pltpu.sync_copy(src3.at[wid, j], src_b)
        pltpu.sync_copy(dst3.at[wid, j], dst_b)
        pltpu.async_copy(table.at[src_b], rows, sem).wait()
        pltpu.sync_copy(rows, agg_sh.at[dst_b], add=True)
        for k in range(C // 16):
            idx = dst_b[pl.ds(k * 16, 16)]
            plsc.addupdate_scatter(deg_v, [idx], ones16)

    plsc.subcore_barrier()

    pltpu.sync_copy(agg_sh.at[pl.ds(base, ROWS_PER_TILE)],
                    agg_out.at[cid, pl.ds(base, ROWS_PER_TILE)])
    pltpu.sync_copy(deg_v, deg_out.at[wid])


@jax.jit
def _sc_agg(table, src3, dst3):
    mesh = plsc.VectorSubcoreMesh(core_axis_name="c", subcore_axis_name="s")
    return pl.kernel(
        _sc_agg_body,
        out_type=(
            jax.ShapeDtypeStruct((NC, N_PAD, D), jnp.float32),
            jax.ShapeDtypeStruct((NW, N_PAD), jnp.float32),
        ),
        mesh=mesh,
        scratch_types=[
            pltpu.VMEM_SHARED((N_PAD, D), jnp.float32),
            pltpu.VMEM((C,), jnp.int32),
            pltpu.VMEM((C,), jnp.int32),
            pltpu.VMEM((C, D), jnp.float32),
            pltpu.VMEM((N_PAD,), jnp.float32),
            pltpu.SemaphoreType.DMA,
        ],
        compiler_params=pltpu.CompilerParams(needs_layout_passes=False),
    )(table, src3, dst3)


def _tc_layer_body(relu, p_ref, deg_ref, x_ref, wl_ref, wr_ref, b_ref, o_ref):
    agg = p_ref[0] + p_ref[1]
    deg = jnp.sum(deg_ref[...], axis=0)
    deg = jnp.maximum(deg, 1.0)
    agg = agg / deg[:, None]
    out = (jnp.dot(agg, wl_ref[...], preferred_element_type=jnp.float32)
           + jnp.dot(x_ref[...], wr_ref[...], preferred_element_type=jnp.float32)
           + b_ref[...])
    if relu:
        out = jnp.maximum(out, 0.0)
    o_ref[...] = out


BLK = 512


@functools.partial(jax.jit, static_argnames=("relu",))
def _tc_layer(P, degP, x, W_l, W_r, b, relu):
    grid = N_PAD // BLK
    return pl.pallas_call(
        functools.partial(_tc_layer_body, relu),
        grid=(grid,),
        in_specs=[
            pl.BlockSpec((NC, BLK, D), lambda i: (0, i, 0)),
            pl.BlockSpec((NW, BLK), lambda i: (0, i)),
            pl.BlockSpec((BLK, D), lambda i: (i, 0)),
            pl.BlockSpec((D, D), lambda i: (0, 0)),
            pl.BlockSpec((D, D), lambda i: (0, 0)),
            pl.BlockSpec((1, D), lambda i: (0, 0)),
        ],
        out_specs=pl.BlockSpec((BLK, D), lambda i: (i, 0)),
        out_shape=jax.ShapeDtypeStruct((N_PAD, D), jnp.float32),
    )(P, degP, x, W_l, W_r, b.reshape(1, D))


def kernel(x, edge_index, W_l1, W_r1, b1, W_l2, W_r2, b2):
    src = edge_index[0]
    dst = edge_index[1]
    e = src.shape[0]
    pad = NW * J * C - e
    src_p = jnp.concatenate([src, jnp.zeros((pad,), jnp.int32)])
    dst_p = jnp.concatenate([dst, jnp.full((pad,), N_NODES, jnp.int32)])
    src3 = src_p.reshape(NW, J, C)
    dst3 = dst_p.reshape(NW, J, C)
    x_pad = jnp.pad(x, ((0, N_PAD - N_NODES), (0, 0)))

    P1, degP = _sc_agg(x_pad, src3, dst3)
    h = _tc_layer(P1, degP, x_pad, W_l1, W_r1, b1, relu=True)
    P2, _ = _sc_agg(h, src3, dst3)
    out = _tc_layer(P2, degP, h, W_l2, W_r2, b2, relu=False)
    return out[:N_NODES]


# speedup vs baseline: 3.6746x; 1.2023x over previous
import functools

import jax
import jax.numpy as jnp
from jax import lax
from jax.experimental import pallas as pl
from jax.experimental.pallas import tpu as pltpu
from jax.experimental.pallas import tpu_sc as plsc

N_NODES = 10000
D = 128
NC = 2
NS = 16
NW = NC * NS
C = 128
J = 80
N_PAD = 10240
ROWS_PER_TILE = N_PAD // NS
DW = 16


G = 16
NG = J // G


def _sc_agg_body(with_deg, table, edge3, *refs):
    if with_deg:
        agg_out, deg_out, agg_sh, idxb, rows, deg_v, gsem0, gsem1 = refs
    else:
        agg_out, agg_sh, idxb, rows, gsem0, gsem1 = refs
    cid = lax.axis_index("c")
    sid = lax.axis_index("s")
    wid = sid * NC + cid

    zeros16 = jnp.zeros((16,), jnp.float32)
    ones16 = jnp.ones((16,), jnp.float32)

    @pl.loop(0, C)
    def _(i):
        for k in range(8):
            rows[0, i, pl.ds(k * 16, 16)] = zeros16

    if with_deg:
        @pl.loop(0, N_PAD // 16)
        def _(i):
            deg_v[pl.ds(i * 16, 16)] = zeros16

    base = sid * ROWS_PER_TILE
    for r in range(ROWS_PER_TILE // C):
        pltpu.sync_copy(rows.at[0], agg_sh.at[pl.ds(base + r * C, C)])

    plsc.subcore_barrier()

    sems = (gsem0, gsem1)

    @pl.loop(0, NG)
    def _(g):
        pltpu.sync_copy(edge3.at[wid, g], idxb)
        pltpu.make_async_copy(table.at[idxb.at[0, 0]], rows.at[0], gsem0).start()
        for u in range(G):
            b = u % 2
            pltpu.make_async_copy(table.at[idxb.at[u, 0]], rows.at[b],
                                  sems[b]).wait()
            if u + 1 < G:
                nb = 1 - b
                pltpu.make_async_copy(table.at[idxb.at[u + 1, 0]], rows.at[nb],
                                      sems[nb]).start()
            pltpu.sync_copy(rows.at[b], agg_sh.at[idxb.at[u, 1]], add=True)
            if with_deg:
                for k in range(C // 16):
                    idx = idxb[u, 1, pl.ds(k * 16, 16)]
                    plsc.addupdate_scatter(deg_v, [idx], ones16)

    plsc.subcore_barrier()

    pltpu.sync_copy(agg_sh.at[pl.ds(base, ROWS_PER_TILE)],
                    agg_out.at[cid, pl.ds(base, ROWS_PER_TILE)])
    if with_deg:
        pltpu.sync_copy(deg_v, deg_out.at[wid])


@functools.partial(jax.jit, static_argnames=("with_deg",))
def _sc_agg(table, edge3, with_deg):
    mesh = plsc.VectorSubcoreMesh(core_axis_name="c", subcore_axis_name="s")
    out_type = [jax.ShapeDtypeStruct((NC, N_PAD, D), jnp.float32)]
    scratch = [
        pltpu.VMEM_SHARED((N_PAD, D), jnp.float32),
        pltpu.VMEM((G, 2, C), jnp.int32),
        pltpu.VMEM((2, C, D), jnp.float32),
    ]
    if with_deg:
        out_type.append(jax.ShapeDtypeStruct((NW, N_PAD), jnp.float32))
        scratch.append(pltpu.VMEM((N_PAD,), jnp.float32))
    scratch += [pltpu.SemaphoreType.DMA, pltpu.SemaphoreType.DMA]
    return pl.kernel(
        functools.partial(_sc_agg_body, with_deg),
        out_type=tuple(out_type),
        mesh=mesh,
        scratch_types=scratch,
        compiler_params=pltpu.CompilerParams(needs_layout_passes=False),
    )(table, edge3)


def _tc_layer_body(relu, p_ref, deg_ref, x_ref, wl_ref, wr_ref, b_ref, o_ref):
    agg = p_ref[0] + p_ref[1]
    deg = jnp.sum(deg_ref[...], axis=0)
    deg = jnp.maximum(deg, 1.0)
    agg = agg / deg[:, None]
    out = (jnp.dot(agg, wl_ref[...], preferred_element_type=jnp.float32)
           + jnp.dot(x_ref[...], wr_ref[...], preferred_element_type=jnp.float32)
           + b_ref[...])
    if relu:
        out = jnp.maximum(out, 0.0)
    o_ref[...] = out


BLK = 512


@functools.partial(jax.jit, static_argnames=("relu",))
def _tc_layer(P, degP, x, W_l, W_r, b, relu):
    grid = N_PAD // BLK
    return pl.pallas_call(
        functools.partial(_tc_layer_body, relu),
        grid=(grid,),
        in_specs=[
            pl.BlockSpec((NC, BLK, D), lambda i: (0, i, 0)),
            pl.BlockSpec((NW, BLK), lambda i: (0, i)),
            pl.BlockSpec((BLK, D), lambda i: (i, 0)),
            pl.BlockSpec((D, D), lambda i: (0, 0)),
            pl.BlockSpec((D, D), lambda i: (0, 0)),
            pl.BlockSpec((1, D), lambda i: (0, 0)),
        ],
        out_specs=pl.BlockSpec((BLK, D), lambda i: (i, 0)),
        out_shape=jax.ShapeDtypeStruct((N_PAD, D), jnp.float32),
    )(P, degP, x, W_l, W_r, b.reshape(1, D))


def kernel(x, edge_index, W_l1, W_r1, b1, W_l2, W_r2, b2):
    src = edge_index[0]
    dst = edge_index[1]
    e = src.shape[0]
    pad = NW * J * C - e
    src_p = jnp.concatenate([src, jnp.zeros((pad,), jnp.int32)])
    dst_p = jnp.concatenate([dst, jnp.full((pad,), N_NODES, jnp.int32)])
    src3 = src_p.reshape(NW, J, C)
    dst3 = dst_p.reshape(NW, J, C)
    edge3 = jnp.stack([src3, dst3], axis=2).reshape(NW, NG, G, 2, C)
    x_pad = jnp.pad(x, ((0, N_PAD - N_NODES), (0, 0)))

    P1, degP = _sc_agg(x_pad, edge3, with_deg=True)
    h = _tc_layer(P1, degP, x_pad, W_l1, W_r1, b1, relu=True)
    (P2,) = _sc_agg(h, edge3, with_deg=False)
    out = _tc_layer(P2, degP, h, W_l2, W_r2, b2, relu=False)
    return out[:N_NODES]
